# baseline (speedup 1.0000x reference)
import functools

import jax
import jax.numpy as jnp
from jax import lax
from jax.experimental import pallas as pl
from jax.experimental.pallas import tpu as pltpu
from jax.experimental.pallas import tpu_sc as plsc

_MB = 256
_CH = 512


def _meta_body(idx_s_ref, idx_t_ref, cm_t_ref, pos_ref, blk_ref, util_ref,
               rank_ref, *, S, E, NBLK):
    eids = lax.broadcasted_iota(jnp.int32, (S, E), 1)
    O = (idx_s_ref[...] == eids).astype(jnp.float32)

    ri = lax.broadcasted_iota(jnp.int32, (_CH, _CH), 0)
    ci = lax.broadcasted_iota(jnp.int32, (_CH, _CH), 1)
    LT = (ri > ci).astype(jnp.float32)
    carry = jnp.zeros((1, E), jnp.float32)
    for i in range(S // _CH):
        Oc = O[i * _CH:(i + 1) * _CH]
        rank_ref[i * _CH:(i + 1) * _CH, :] = (
            jnp.dot(LT, Oc, preferred_element_type=jnp.float32) + carry)
        carry = carry + jnp.sum(Oc, axis=0, keepdims=True)
    counts = carry
    padded = jnp.floor((counts + (_MB - 1)) * (1.0 / _MB)) * _MB

    eri = lax.broadcasted_iota(jnp.int32, (E, E), 0)
    eci = lax.broadcasted_iota(jnp.int32, (E, E), 1)
    UT = (eri < eci).astype(jnp.float32)
    bstart = jnp.dot(padded, UT, preferred_element_type=jnp.float32)

    posf = jnp.sum((rank_ref[...] + bstart) * O, axis=1, keepdims=True)
    pos_ref[...] = posf.astype(jnp.int32)

    bpos = lax.broadcasted_iota(jnp.int32, (NBLK, 1), 0).astype(jnp.float32) * _MB
    cnt = jnp.sum((bstart <= bpos).astype(jnp.float32), axis=1, keepdims=True)
    total = jnp.sum(padded)
    blk_ref[...] = jnp.where(bpos < total, cnt - 1.0, -1.0).astype(jnp.int32)

    eids_t = lax.broadcasted_iota(jnp.int32, (idx_t_ref.shape[0], E), 1)
    acc = jnp.zeros((idx_t_ref.shape[0], E), jnp.float32)
    for k in range(idx_t_ref.shape[1]):
        hit = (idx_t_ref[:, k:k + 1] == eids_t) & (cm_t_ref[:, k:k + 1] > 0.0)
        acc = jnp.maximum(acc, hit.astype(jnp.float32))
    util_ref[...] = acc


def _wcast_body(W1_ref, W2_ref, W1o_ref, W2o_ref):
    W1o_ref[...] = W1_ref[...].astype(jnp.bfloat16)
    W2o_ref[...] = W2_ref[...].astype(jnp.bfloat16)


def _mlp_body(blk_ref, xs_ref, W1_ref, b1_ref, W2_ref, b2_ref, z_ref):
    b = pl.program_id(0)

    @pl.when(blk_ref[b] >= 0)
    def _():
        xb = xs_ref[...].astype(jnp.bfloat16)
        h = jnp.maximum(
            jnp.dot(xb, W1_ref[0], preferred_element_type=jnp.float32)
            + b1_ref[0], 0.0)
        z_ref[...] = jnp.dot(h.astype(jnp.bfloat16), W2_ref[0],
                             preferred_element_type=jnp.float32) + b2_ref[0]

    @pl.when(blk_ref[b] < 0)
    def _():
        z_ref[...] = jnp.zeros_like(z_ref)


def _combine_body(x_ref, w_ref, cm_ref, *rest, T):
    K = w_ref.shape[1]
    z_refs = rest[:K]
    scale_ref = rest[K]
    out_ref = rest[K + 1]
    wb = w_ref[...]
    cmb = cm_ref[...]
    scale = scale_ref[0, 0]
    wsum = jnp.clip(jnp.sum(wb, axis=1, keepdims=True), 1e-12, None)
    nw = wb / wsum
    acc = jnp.sum(nw, axis=1, keepdims=True) * x_ref[...]
    for k in range(K):
        acc = acc + (nw[:, k:k + 1] * cmb[:, k:k + 1]) * z_refs[k][...]
    out_ref[...] = scale * acc


def kernel(x, chosen_idx, chosen_w, compute_mask, route_scale, W1, b1, W2, b2):
    B, N, T = x.shape
    K = chosen_idx.shape[-1]
    E, _, H = W1.shape
    BN = B * N
    S = BN * K
    NBLK = S // _MB + E
    RMAX = NBLK * _MB
    NC, NS = 2, 16
    NW = NC * NS
    SPW = S // NW

    x_flat = x.reshape(BN, T)
    idx_s = chosen_idx.reshape(S, 1).astype(jnp.int32)
    idx_t = chosen_idx.reshape(BN, K).astype(jnp.int32)
    w_t = chosen_w.reshape(BN, K)
    cm_t = compute_mask.reshape(BN, K).astype(jnp.float32)
    scale = route_scale.reshape(1, 1).astype(jnp.float32)
    tok_of_slot = (jnp.arange(S, dtype=jnp.int32) // K).reshape(NW, SPW)

    pos, blk, util = pl.pallas_call(
        functools.partial(_meta_body, S=S, E=E, NBLK=NBLK),
        out_shape=[
            jax.ShapeDtypeStruct((S, 1), jnp.int32),
            jax.ShapeDtypeStruct((NBLK, 1), jnp.int32),
            jax.ShapeDtypeStruct((BN, E), jnp.float32),
        ],
        scratch_shapes=[pltpu.VMEM((S, E), jnp.float32)],
    )(idx_s, idx_t, cm_t)
    pos_rows = pos.reshape(NW, SPW)

    mesh = plsc.VectorSubcoreMesh(core_axis_name="c", subcore_axis_name="s",
                                  num_cores=NC, num_subcores=NS)

    @functools.partial(
        pl.kernel,
        out_type=jax.ShapeDtypeStruct((RMAX, T), jnp.float32),
        mesh=mesh,
        scratch_types=[
            pltpu.VMEM((SPW,), jnp.int32),
            pltpu.VMEM((SPW,), jnp.int32),
            pltpu.VMEM((SPW, T), jnp.float32),
            pltpu.SemaphoreType.DMA,
            pltpu.SemaphoreType.DMA,
        ],
    )
    def dispatch(x_hbm, tok_hbm, pos_hbm, xs_hbm, tok_v, pos_v, xbuf, s1, s2):
        wid = lax.axis_index("s") * NC + lax.axis_index("c")
        pltpu.sync_copy(tok_hbm.at[wid], tok_v)
        pltpu.sync_copy(pos_hbm.at[wid], pos_v)
        pltpu.async_copy(x_hbm.at[tok_v], xbuf, s1).wait()
        pltpu.async_copy(xbuf, xs_hbm.at[pos_v], s2).wait()

    xs = dispatch(x_flat, tok_of_slot, pos_rows)

    WSP = 4
    W1b, W2b = pl.pallas_call(
        _wcast_body,
        grid=(E, WSP),
        in_specs=[
            pl.BlockSpec((1, T // WSP, H), lambda e, j: (e, j, 0)),
            pl.BlockSpec((1, H // WSP, T), lambda e, j: (e, j, 0)),
        ],
        out_specs=[
            pl.BlockSpec((1, T // WSP, H), lambda e, j: (e, j, 0)),
            pl.BlockSpec((1, H // WSP, T), lambda e, j: (e, j, 0)),
        ],
        out_shape=[
            jax.ShapeDtypeStruct((E, T, H), jnp.bfloat16),
            jax.ShapeDtypeStruct((E, H, T), jnp.bfloat16),
        ],
    )(W1, W2)

    z_sorted = pl.pallas_call(
        _mlp_body,
        grid_spec=pltpu.PrefetchScalarGridSpec(
            num_scalar_prefetch=1,
            grid=(NBLK,),
            in_specs=[
                pl.BlockSpec((_MB, T), lambda b, blk: (b, 0)),
                pl.BlockSpec((1, T, H), lambda b, blk: (jnp.maximum(blk[b], 0), 0, 0)),
                pl.BlockSpec((1, 1, H), lambda b, blk: (jnp.maximum(blk[b], 0), 0, 0)),
                pl.BlockSpec((1, H, T), lambda b, blk: (jnp.maximum(blk[b], 0), 0, 0)),
                pl.BlockSpec((1, 1, T), lambda b, blk: (jnp.maximum(blk[b], 0), 0, 0)),
            ],
            out_specs=pl.BlockSpec((_MB, T), lambda b, blk: (b, 0)),
        ),
        out_shape=jax.ShapeDtypeStruct((RMAX, T), jnp.float32),
        compiler_params=pltpu.CompilerParams(
            dimension_semantics=("arbitrary",),
        ),
    )(blk.reshape(NBLK), xs,
      W1b, b1.reshape(E, 1, H),
      W2b, b2.reshape(E, 1, T))

    TPW = BN // NW
    posK = pos.reshape(BN, K)
    pos_k = [posK[:, k].reshape(NW, TPW) for k in range(K)]

    @functools.partial(
        pl.kernel,
        out_type=[jax.ShapeDtypeStruct((BN, T), jnp.float32) for _ in range(K)],
        mesh=mesh,
        scratch_types=(
            [pltpu.VMEM((TPW,), jnp.int32) for _ in range(K)]
            + [pltpu.VMEM((TPW, T), jnp.float32) for _ in range(K)]
            + [pltpu.SemaphoreType.DMA]
        ),
    )
    def unsort(z_hbm, *refs):
        pos_hbms = refs[:K]
        z_outs = refs[K:2 * K]
        pos_vs = refs[2 * K:3 * K]
        zbufs = refs[3 * K:4 * K]
        s1 = refs[4 * K]
        wid = lax.axis_index("s") * NC + lax.axis_index("c")
        for k in range(K):
            pltpu.sync_copy(pos_hbms[k].at[wid], pos_vs[k])
            pltpu.async_copy(z_hbm.at[pos_vs[k]], zbufs[k], s1).wait()
            pltpu.sync_copy(zbufs[k], z_outs[k].at[pl.ds(wid * TPW, TPW)])

    z_k = unsort(z_sorted, *pos_k)

    TBc = 256
    out = pl.pallas_call(
        functools.partial(_combine_body, T=T),
        grid=(BN // TBc,),
        in_specs=(
            [
                pl.BlockSpec((TBc, T), lambda i: (i, 0)),
                pl.BlockSpec((TBc, K), lambda i: (i, 0)),
                pl.BlockSpec((TBc, K), lambda i: (i, 0)),
            ]
            + [pl.BlockSpec((TBc, T), lambda i: (i, 0)) for _ in range(K)]
            + [pl.BlockSpec(memory_space=pltpu.SMEM)]
        ),
        out_specs=pl.BlockSpec((TBc, T), lambda i: (i, 0)),
        out_shape=jax.ShapeDtypeStruct((BN, T), jnp.float32),
    )(x_flat, w_t, cm_t, *z_k, scale)

    return out.reshape(B, N, T), util.reshape(B, N, E)


# speedup vs baseline: 1.0130x; 1.0130x over previous
import functools

import jax
import jax.numpy as jnp
from jax import lax
from jax.experimental import pallas as pl
from jax.experimental.pallas import tpu as pltpu
from jax.experimental.pallas import tpu_sc as plsc

_MB = 256
_CH = 512


def _meta_body(idx_s_ref, idx_t_ref, cm_t_ref, pos_ref, blk_ref, util_ref,
               rank_ref, *, S, E, NBLK):
    eids = lax.broadcasted_iota(jnp.int32, (S, E), 1)
    O = (idx_s_ref[...] == eids).astype(jnp.float32)

    ri = lax.broadcasted_iota(jnp.int32, (_CH, _CH), 0)
    ci = lax.broadcasted_iota(jnp.int32, (_CH, _CH), 1)
    LT = (ri > ci).astype(jnp.float32)
    carry = jnp.zeros((1, E), jnp.float32)
    for i in range(S // _CH):
        Oc = O[i * _CH:(i + 1) * _CH]
        rank_ref[i * _CH:(i + 1) * _CH, :] = (
            jnp.dot(LT, Oc, preferred_element_type=jnp.float32) + carry)
        carry = carry + jnp.sum(Oc, axis=0, keepdims=True)
    counts = carry
    padded = jnp.floor((counts + (_MB - 1)) * (1.0 / _MB)) * _MB

    eri = lax.broadcasted_iota(jnp.int32, (E, E), 0)
    eci = lax.broadcasted_iota(jnp.int32, (E, E), 1)
    UT = (eri < eci).astype(jnp.float32)
    bstart = jnp.dot(padded, UT, preferred_element_type=jnp.float32)

    posf = jnp.sum((rank_ref[...] + bstart) * O, axis=1, keepdims=True)
    pos_ref[...] = posf.astype(jnp.int32)

    bpos = lax.broadcasted_iota(jnp.int32, (NBLK, 1), 0).astype(jnp.float32) * _MB
    cnt = jnp.sum((bstart <= bpos).astype(jnp.float32), axis=1, keepdims=True)
    total = jnp.sum(padded)
    blk_ref[...] = jnp.where(bpos < total, cnt - 1.0, -1.0).astype(jnp.int32)

    eids_t = lax.broadcasted_iota(jnp.int32, (idx_t_ref.shape[0], E), 1)
    acc = jnp.zeros((idx_t_ref.shape[0], E), jnp.float32)
    for k in range(idx_t_ref.shape[1]):
        hit = (idx_t_ref[:, k:k + 1] == eids_t) & (cm_t_ref[:, k:k + 1] > 0.0)
        acc = jnp.maximum(acc, hit.astype(jnp.float32))
    util_ref[...] = acc


def _wcast_body(W1_ref, W2_ref, W1o_ref, W2o_ref):
    W1o_ref[...] = W1_ref[...].astype(jnp.bfloat16)
    W2o_ref[...] = W2_ref[...].astype(jnp.bfloat16)


def _mlp_body(blk_ref, xs_ref, W1_ref, b1_ref, W2_ref, b2_ref, z_ref):
    b = pl.program_id(0)

    @pl.when(blk_ref[b] >= 0)
    def _():
        xb = xs_ref[...].astype(jnp.bfloat16)
        h = jnp.maximum(
            jnp.dot(xb, W1_ref[0], preferred_element_type=jnp.float32)
            + b1_ref[0], 0.0)
        z_ref[...] = jnp.dot(h.astype(jnp.bfloat16), W2_ref[0],
                             preferred_element_type=jnp.float32) + b2_ref[0]

    @pl.when(blk_ref[b] < 0)
    def _():
        z_ref[...] = jnp.zeros_like(z_ref)


def _combine_body(x_ref, w_ref, cm_ref, *rest, T):
    K = w_ref.shape[1]
    z_refs = rest[:K]
    scale_ref = rest[K]
    out_ref = rest[K + 1]
    wb = w_ref[...]
    cmb = cm_ref[...]
    scale = scale_ref[0, 0]
    wsum = jnp.clip(jnp.sum(wb, axis=1, keepdims=True), 1e-12, None)
    nw = wb / wsum
    acc = jnp.sum(nw, axis=1, keepdims=True) * x_ref[...]
    for k in range(K):
        acc = acc + (nw[:, k:k + 1] * cmb[:, k:k + 1]) * z_refs[k][...]
    out_ref[...] = scale * acc


def kernel(x, chosen_idx, chosen_w, compute_mask, route_scale, W1, b1, W2, b2):
    B, N, T = x.shape
    K = chosen_idx.shape[-1]
    E, _, H = W1.shape
    BN = B * N
    S = BN * K
    NBLK = S // _MB + E
    RMAX = NBLK * _MB
    NC, NS = 2, 16
    NW = NC * NS
    SPW = S // NW

    x_flat = x.reshape(BN, T)
    idx_s = chosen_idx.reshape(S, 1).astype(jnp.int32)
    idx_t = chosen_idx.reshape(BN, K).astype(jnp.int32)
    w_t = chosen_w.reshape(BN, K)
    cm_t = compute_mask.reshape(BN, K).astype(jnp.float32)
    scale = route_scale.reshape(1, 1).astype(jnp.float32)
    tok_of_slot = (jnp.arange(S, dtype=jnp.int32) // K).reshape(NW, SPW)

    pos, blk, util = pl.pallas_call(
        functools.partial(_meta_body, S=S, E=E, NBLK=NBLK),
        out_shape=[
            jax.ShapeDtypeStruct((S, 1), jnp.int32),
            jax.ShapeDtypeStruct((NBLK, 1), jnp.int32),
            jax.ShapeDtypeStruct((BN, E), jnp.float32),
        ],
        scratch_shapes=[pltpu.VMEM((S, E), jnp.float32)],
    )(idx_s, idx_t, cm_t)
    pos_rows = pos.reshape(NW, SPW)

    mesh = plsc.VectorSubcoreMesh(core_axis_name="c", subcore_axis_name="s",
                                  num_cores=NC, num_subcores=NS)

    @functools.partial(
        pl.kernel,
        out_type=jax.ShapeDtypeStruct((RMAX, T), jnp.float32),
        mesh=mesh,
        scratch_types=[
            pltpu.VMEM((SPW,), jnp.int32),
            pltpu.VMEM((SPW,), jnp.int32),
            pltpu.VMEM((SPW, T), jnp.float32),
            pltpu.SemaphoreType.DMA,
            pltpu.SemaphoreType.DMA,
        ],
    )
    def dispatch(x_hbm, tok_hbm, pos_hbm, xs_hbm, tok_v, pos_v, xbuf, s1, s2):
        wid = lax.axis_index("s") * NC + lax.axis_index("c")
        pltpu.sync_copy(tok_hbm.at[wid], tok_v)
        pltpu.sync_copy(pos_hbm.at[wid], pos_v)
        pltpu.async_copy(x_hbm.at[tok_v], xbuf, s1).wait()
        pltpu.async_copy(xbuf, xs_hbm.at[pos_v], s2).wait()

    xs = dispatch(x_flat, tok_of_slot, pos_rows)

    WSP = 4
    W1b, W2b = pl.pallas_call(
        _wcast_body,
        grid=(E, WSP),
        in_specs=[
            pl.BlockSpec((1, T // WSP, H), lambda e, j: (e, j, 0)),
            pl.BlockSpec((1, H // WSP, T), lambda e, j: (e, j, 0)),
        ],
        out_specs=[
            pl.BlockSpec((1, T // WSP, H), lambda e, j: (e, j, 0)),
            pl.BlockSpec((1, H // WSP, T), lambda e, j: (e, j, 0)),
        ],
        out_shape=[
            jax.ShapeDtypeStruct((E, T, H), jnp.bfloat16),
            jax.ShapeDtypeStruct((E, H, T), jnp.bfloat16),
        ],
    )(W1, W2)

    z_sorted = pl.pallas_call(
        _mlp_body,
        grid_spec=pltpu.PrefetchScalarGridSpec(
            num_scalar_prefetch=1,
            grid=(NBLK,),
            in_specs=[
                pl.BlockSpec((_MB, T), lambda b, blk: (b, 0)),
                pl.BlockSpec((1, T, H), lambda b, blk: (jnp.maximum(blk[b], 0), 0, 0)),
                pl.BlockSpec((1, 1, H), lambda b, blk: (jnp.maximum(blk[b], 0), 0, 0)),
                pl.BlockSpec((1, H, T), lambda b, blk: (jnp.maximum(blk[b], 0), 0, 0)),
                pl.BlockSpec((1, 1, T), lambda b, blk: (jnp.maximum(blk[b], 0), 0, 0)),
            ],
            out_specs=pl.BlockSpec((_MB, T), lambda b, blk: (b, 0)),
        ),
        out_shape=jax.ShapeDtypeStruct((RMAX, T), jnp.float32),
        compiler_params=pltpu.CompilerParams(
            dimension_semantics=("arbitrary",),
        ),
    )(blk.reshape(NBLK), xs,
      W1b, b1.reshape(E, 1, H),
      W2b, b2.reshape(E, 1, T))

    TPW = BN // NW
    posK = pos.reshape(BN, K)
    pos_k = [posK[:, k].reshape(NW, TPW) for k in range(K)]

    @functools.partial(
        pl.kernel,
        out_type=[jax.ShapeDtypeStruct((BN, T), jnp.float32) for _ in range(K)],
        mesh=mesh,
        scratch_types=(
            [pltpu.VMEM((TPW,), jnp.int32) for _ in range(K)]
            + [pltpu.VMEM((TPW, T), jnp.float32) for _ in range(K)]
            + [pltpu.SemaphoreType.DMA]
        ),
    )
    def unsort(z_hbm, *refs):
        pos_hbms = refs[:K]
        z_outs = refs[K:2 * K]
        pos_vs = refs[2 * K:3 * K]
        zbufs = refs[3 * K:4 * K]
        s1 = refs[4 * K]
        wid = lax.axis_index("s") * NC + lax.axis_index("c")
        for k in range(K):
            pltpu.sync_copy(pos_hbms[k].at[wid], pos_vs[k])
            pltpu.async_copy(z_hbm.at[pos_vs[k]], zbufs[k], s1).wait()
            pltpu.sync_copy(zbufs[k], z_outs[k].at[pl.ds(wid * TPW, TPW)])

    z_k = unsort(z_sorted, *pos_k)

    TBc = 512
    out = pl.pallas_call(
        functools.partial(_combine_body, T=T),
        grid=(BN // TBc,),
        in_specs=(
            [
                pl.BlockSpec((TBc, T), lambda i: (i, 0)),
                pl.BlockSpec((TBc, K), lambda i: (i, 0)),
                pl.BlockSpec((TBc, K), lambda i: (i, 0)),
            ]
            + [pl.BlockSpec((TBc, T), lambda i: (i, 0)) for _ in range(K)]
            + [pl.BlockSpec(memory_space=pltpu.SMEM)]
        ),
        out_specs=pl.BlockSpec((TBc, T), lambda i: (i, 0)),
        out_shape=jax.ShapeDtypeStruct((BN, T), jnp.float32),
    )(x_flat, w_t, cm_t, *z_k, scale)

    return out.reshape(B, N, T), util.reshape(B, N, E)


# speedup vs baseline: 1.1009x; 1.0868x over previous
import functools

import jax
import jax.numpy as jnp
from jax import lax
from jax.experimental import pallas as pl
from jax.experimental.pallas import tpu as pltpu
from jax.experimental.pallas import tpu_sc as plsc

_MB = 256
_CH = 512


def _meta_body(idx_s_ref, idx_t_ref, cm_t_ref, pos_ref, blk_ref, util_ref,
               rank_ref, *, S, E, NBLK):
    eids = lax.broadcasted_iota(jnp.int32, (S, E), 1)
    O = (idx_s_ref[...] == eids).astype(jnp.float32)

    ri = lax.broadcasted_iota(jnp.int32, (_CH, _CH), 0)
    ci = lax.broadcasted_iota(jnp.int32, (_CH, _CH), 1)
    LT = (ri > ci).astype(jnp.float32)
    carry = jnp.zeros((1, E), jnp.float32)
    for i in range(S // _CH):
        Oc = O[i * _CH:(i + 1) * _CH]
        rank_ref[i * _CH:(i + 1) * _CH, :] = (
            jnp.dot(LT, Oc, preferred_element_type=jnp.float32) + carry)
        carry = carry + jnp.sum(Oc, axis=0, keepdims=True)
    counts = carry
    padded = jnp.floor((counts + (_MB - 1)) * (1.0 / _MB)) * _MB

    eri = lax.broadcasted_iota(jnp.int32, (E, E), 0)
    eci = lax.broadcasted_iota(jnp.int32, (E, E), 1)
    UT = (eri < eci).astype(jnp.float32)
    bstart = jnp.dot(padded, UT, preferred_element_type=jnp.float32)

    posf = jnp.sum((rank_ref[...] + bstart) * O, axis=1, keepdims=True)
    pos_ref[...] = posf.astype(jnp.int32)

    bpos = lax.broadcasted_iota(jnp.int32, (NBLK, 1), 0).astype(jnp.float32) * _MB
    cnt = jnp.sum((bstart <= bpos).astype(jnp.float32), axis=1, keepdims=True)
    total = jnp.sum(padded)
    blk_ref[...] = jnp.where(bpos < total, cnt - 1.0, -1.0).astype(jnp.int32)

    eids_t = lax.broadcasted_iota(jnp.int32, (idx_t_ref.shape[0], E), 1)
    acc = jnp.zeros((idx_t_ref.shape[0], E), jnp.float32)
    for k in range(idx_t_ref.shape[1]):
        hit = (idx_t_ref[:, k:k + 1] == eids_t) & (cm_t_ref[:, k:k + 1] > 0.0)
        acc = jnp.maximum(acc, hit.astype(jnp.float32))
    util_ref[...] = acc


def _wcast_body(W1_ref, W2_ref, W1o_ref, W2o_ref):
    W1o_ref[...] = W1_ref[...].astype(jnp.bfloat16)
    W2o_ref[...] = W2_ref[...].astype(jnp.bfloat16)


def _mlp_body(blk_ref, xs_ref, W1_ref, b1_ref, W2_ref, b2_ref, z_ref):
    b = pl.program_id(0)

    @pl.when(blk_ref[b] >= 0)
    def _():
        e = jnp.maximum(blk_ref[b], 0)
        xb = xs_ref[...].astype(jnp.bfloat16)
        h = jnp.maximum(
            jnp.dot(xb, W1_ref[e], preferred_element_type=jnp.float32)
            + b1_ref[e], 0.0)
        z_ref[...] = jnp.dot(h.astype(jnp.bfloat16), W2_ref[e],
                             preferred_element_type=jnp.float32) + b2_ref[e]

    @pl.when(blk_ref[b] < 0)
    def _():
        z_ref[...] = jnp.zeros_like(z_ref)


def _combine_body(x_ref, w_ref, cm_ref, *rest, T):
    K = w_ref.shape[1]
    z_refs = rest[:K]
    scale_ref = rest[K]
    out_ref = rest[K + 1]
    wb = w_ref[...]
    cmb = cm_ref[...]
    scale = scale_ref[0, 0]
    wsum = jnp.clip(jnp.sum(wb, axis=1, keepdims=True), 1e-12, None)
    nw = wb / wsum
    acc = jnp.sum(nw, axis=1, keepdims=True) * x_ref[...]
    for k in range(K):
        acc = acc + (nw[:, k:k + 1] * cmb[:, k:k + 1]) * z_refs[k][...]
    out_ref[...] = scale * acc


def kernel(x, chosen_idx, chosen_w, compute_mask, route_scale, W1, b1, W2, b2):
    B, N, T = x.shape
    K = chosen_idx.shape[-1]
    E, _, H = W1.shape
    BN = B * N
    S = BN * K
    NBLK = S // _MB + E
    RMAX = NBLK * _MB
    NC, NS = 2, 16
    NW = NC * NS
    SPW = S // NW

    x_flat = x.reshape(BN, T)
    idx_s = chosen_idx.reshape(S, 1).astype(jnp.int32)
    idx_t = chosen_idx.reshape(BN, K).astype(jnp.int32)
    w_t = chosen_w.reshape(BN, K)
    cm_t = compute_mask.reshape(BN, K).astype(jnp.float32)
    scale = route_scale.reshape(1, 1).astype(jnp.float32)
    tok_of_slot = (jnp.arange(S, dtype=jnp.int32) // K).reshape(NW, SPW)

    pos, blk, util = pl.pallas_call(
        functools.partial(_meta_body, S=S, E=E, NBLK=NBLK),
        out_shape=[
            jax.ShapeDtypeStruct((S, 1), jnp.int32),
            jax.ShapeDtypeStruct((NBLK, 1), jnp.int32),
            jax.ShapeDtypeStruct((BN, E), jnp.float32),
        ],
        scratch_shapes=[pltpu.VMEM((S, E), jnp.float32)],
    )(idx_s, idx_t, cm_t)
    pos_rows = pos.reshape(NW, SPW)

    mesh = plsc.VectorSubcoreMesh(core_axis_name="c", subcore_axis_name="s",
                                  num_cores=NC, num_subcores=NS)

    @functools.partial(
        pl.kernel,
        out_type=jax.ShapeDtypeStruct((RMAX, T), jnp.float32),
        mesh=mesh,
        scratch_types=[
            pltpu.VMEM((SPW,), jnp.int32),
            pltpu.VMEM((SPW,), jnp.int32),
            pltpu.VMEM((SPW, T), jnp.float32),
            pltpu.SemaphoreType.DMA,
            pltpu.SemaphoreType.DMA,
        ],
    )
    def dispatch(x_hbm, tok_hbm, pos_hbm, xs_hbm, tok_v, pos_v, xbuf, s1, s2):
        wid = lax.axis_index("s") * NC + lax.axis_index("c")
        pltpu.sync_copy(tok_hbm.at[wid], tok_v)
        pltpu.sync_copy(pos_hbm.at[wid], pos_v)
        pltpu.async_copy(x_hbm.at[tok_v], xbuf, s1).wait()
        pltpu.async_copy(xbuf, xs_hbm.at[pos_v], s2).wait()

    xs = dispatch(x_flat, tok_of_slot, pos_rows)

    W1b, W2b = pl.pallas_call(
        _wcast_body,
        grid=(E,),
        in_specs=[
            pl.BlockSpec((1, T, H), lambda e: (e, 0, 0)),
            pl.BlockSpec((1, H, T), lambda e: (e, 0, 0)),
        ],
        out_specs=[
            pl.BlockSpec((1, T, H), lambda e: (e, 0, 0)),
            pl.BlockSpec((1, H, T), lambda e: (e, 0, 0)),
        ],
        out_shape=[
            jax.ShapeDtypeStruct((E, T, H), jnp.bfloat16),
            jax.ShapeDtypeStruct((E, H, T), jnp.bfloat16),
        ],
    )(W1, W2)

    z_sorted = pl.pallas_call(
        _mlp_body,
        grid_spec=pltpu.PrefetchScalarGridSpec(
            num_scalar_prefetch=1,
            grid=(NBLK,),
            in_specs=[
                pl.BlockSpec((_MB, T), lambda b, blk: (b, 0)),
                pl.BlockSpec((E, T, H), lambda b, blk: (0, 0, 0)),
                pl.BlockSpec((E, 1, H), lambda b, blk: (0, 0, 0)),
                pl.BlockSpec((E, H, T), lambda b, blk: (0, 0, 0)),
                pl.BlockSpec((E, 1, T), lambda b, blk: (0, 0, 0)),
            ],
            out_specs=pl.BlockSpec((_MB, T), lambda b, blk: (b, 0)),
        ),
        out_shape=jax.ShapeDtypeStruct((RMAX, T), jnp.float32),
        compiler_params=pltpu.CompilerParams(
            dimension_semantics=("arbitrary",),
        ),
    )(blk.reshape(NBLK), xs,
      W1b, b1.reshape(E, 1, H),
      W2b, b2.reshape(E, 1, T))

    TPW = BN // NW
    posK = pos.reshape(BN, K)
    pos_k = [posK[:, k].reshape(NW, TPW) for k in range(K)]

    @functools.partial(
        pl.kernel,
        out_type=[jax.ShapeDtypeStruct((BN, T), jnp.float32) for _ in range(K)],
        mesh=mesh,
        scratch_types=(
            [pltpu.VMEM((TPW,), jnp.int32) for _ in range(K)]
            + [pltpu.VMEM((TPW, T), jnp.float32) for _ in range(K)]
            + [pltpu.SemaphoreType.DMA]
        ),
    )
    def unsort(z_hbm, *refs):
        pos_hbms = refs[:K]
        z_outs = refs[K:2 * K]
        pos_vs = refs[2 * K:3 * K]
        zbufs = refs[3 * K:4 * K]
        s1 = refs[4 * K]
        wid = lax.axis_index("s") * NC + lax.axis_index("c")
        for k in range(K):
            pltpu.sync_copy(pos_hbms[k].at[wid], pos_vs[k])
            pltpu.async_copy(z_hbm.at[pos_vs[k]], zbufs[k], s1).wait()
            pltpu.sync_copy(zbufs[k], z_outs[k].at[pl.ds(wid * TPW, TPW)])

    z_k = unsort(z_sorted, *pos_k)

    TBc = 512
    out = pl.pallas_call(
        functools.partial(_combine_body, T=T),
        grid=(BN // TBc,),
        in_specs=(
            [
                pl.BlockSpec((TBc, T), lambda i: (i, 0)),
                pl.BlockSpec((TBc, K), lambda i: (i, 0)),
                pl.BlockSpec((TBc, K), lambda i: (i, 0)),
            ]
            + [pl.BlockSpec((TBc, T), lambda i: (i, 0)) for _ in range(K)]
            + [pl.BlockSpec(memory_space=pltpu.SMEM)]
        ),
        out_specs=pl.BlockSpec((TBc, T), lambda i: (i, 0)),
        out_shape=jax.ShapeDtypeStruct((BN, T), jnp.float32),
    )(x_flat, w_t, cm_t, *z_k, scale)

    return out.reshape(B, N, T), util.reshape(B, N, E)


# speedup vs baseline: 1.1474x; 1.0423x over previous
import functools

import jax
import jax.numpy as jnp
from jax import lax
from jax.experimental import pallas as pl
from jax.experimental.pallas import tpu as pltpu
from jax.experimental.pallas import tpu_sc as plsc

_MB = 256
_CH = 512


def _meta_body(idx_s_ref, idx_t_ref, cm_t_ref, pos_ref, blk_ref, util_ref,
               rank_ref, *, S, E, NBLK):
    eids = lax.broadcasted_iota(jnp.int32, (S, E), 1)
    O = (idx_s_ref[...] == eids).astype(jnp.float32)

    ri = lax.broadcasted_iota(jnp.int32, (_CH, _CH), 0)
    ci = lax.broadcasted_iota(jnp.int32, (_CH, _CH), 1)
    LT = (ri > ci).astype(jnp.float32)
    carry = jnp.zeros((1, E), jnp.float32)
    for i in range(S // _CH):
        Oc = O[i * _CH:(i + 1) * _CH]
        rank_ref[i * _CH:(i + 1) * _CH, :] = (
            jnp.dot(LT, Oc, preferred_element_type=jnp.float32) + carry)
        carry = carry + jnp.sum(Oc, axis=0, keepdims=True)
    counts = carry
    padded = jnp.floor((counts + (_MB - 1)) * (1.0 / _MB)) * _MB

    eri = lax.broadcasted_iota(jnp.int32, (E, E), 0)
    eci = lax.broadcasted_iota(jnp.int32, (E, E), 1)
    UT = (eri < eci).astype(jnp.float32)
    bstart = jnp.dot(padded, UT, preferred_element_type=jnp.float32)

    posf = jnp.sum((rank_ref[...] + bstart) * O, axis=1, keepdims=True)
    pos_ref[...] = posf.astype(jnp.int32)

    bpos = lax.broadcasted_iota(jnp.int32, (NBLK, 1), 0).astype(jnp.float32) * _MB
    cnt = jnp.sum((bstart <= bpos).astype(jnp.float32), axis=1, keepdims=True)
    total = jnp.sum(padded)
    blk_ref[...] = jnp.where(bpos < total, cnt - 1.0, -1.0).astype(jnp.int32)

    eids_t = lax.broadcasted_iota(jnp.int32, (idx_t_ref.shape[0], E), 1)
    acc = jnp.zeros((idx_t_ref.shape[0], E), jnp.float32)
    for k in range(idx_t_ref.shape[1]):
        hit = (idx_t_ref[:, k:k + 1] == eids_t) & (cm_t_ref[:, k:k + 1] > 0.0)
        acc = jnp.maximum(acc, hit.astype(jnp.float32))
    util_ref[...] = acc


def _wcast_body(W1_ref, W2_ref, W1o_ref, W2o_ref):
    W1o_ref[...] = W1_ref[...].astype(jnp.bfloat16)
    W2o_ref[...] = W2_ref[...].astype(jnp.bfloat16)


def _mlp_body(blk_ref, xs_ref, W1_ref, b1_ref, W2_ref, b2_ref, z_ref, *, BPS):
    b = pl.program_id(0)
    for half in range(BPS):
        blk_id = blk_ref[b * BPS + half]
        lo = half * _MB

        @pl.when(blk_id >= 0)
        def _(blk_id=blk_id, lo=lo):
            e = jnp.maximum(blk_id, 0)
            xb = xs_ref[lo:lo + _MB, :].astype(jnp.bfloat16)
            h = jnp.maximum(
                jnp.dot(xb, W1_ref[e], preferred_element_type=jnp.float32)
                + b1_ref[e], 0.0)
            z_ref[lo:lo + _MB, :] = jnp.dot(
                h.astype(jnp.bfloat16), W2_ref[e],
                preferred_element_type=jnp.float32) + b2_ref[e]

        @pl.when(blk_id < 0)
        def _(lo=lo):
            z_ref[lo:lo + _MB, :] = jnp.zeros((_MB, z_ref.shape[1]), z_ref.dtype)


def _combine_body(x_ref, w_ref, cm_ref, *rest, T):
    K = w_ref.shape[1]
    z_refs = rest[:K]
    scale_ref = rest[K]
    out_ref = rest[K + 1]
    wb = w_ref[...]
    cmb = cm_ref[...]
    scale = scale_ref[0, 0]
    wsum = jnp.clip(jnp.sum(wb, axis=1, keepdims=True), 1e-12, None)
    nw = wb / wsum
    acc = jnp.sum(nw, axis=1, keepdims=True) * x_ref[...]
    for k in range(K):
        acc = acc + (nw[:, k:k + 1] * cmb[:, k:k + 1]) * z_refs[k][...]
    out_ref[...] = scale * acc


def kernel(x, chosen_idx, chosen_w, compute_mask, route_scale, W1, b1, W2, b2):
    B, N, T = x.shape
    K = chosen_idx.shape[-1]
    E, _, H = W1.shape
    BN = B * N
    S = BN * K
    NBLK = S // _MB + E
    RMAX = NBLK * _MB
    NC, NS = 2, 16
    NW = NC * NS
    SPW = S // NW

    x_flat = x.reshape(BN, T)
    idx_s = chosen_idx.reshape(S, 1).astype(jnp.int32)
    idx_t = chosen_idx.reshape(BN, K).astype(jnp.int32)
    w_t = chosen_w.reshape(BN, K)
    cm_t = compute_mask.reshape(BN, K).astype(jnp.float32)
    scale = route_scale.reshape(1, 1).astype(jnp.float32)
    tok_of_slot = (jnp.arange(S, dtype=jnp.int32) // K).reshape(NW, SPW)

    pos, blk, util = pl.pallas_call(
        functools.partial(_meta_body, S=S, E=E, NBLK=NBLK),
        out_shape=[
            jax.ShapeDtypeStruct((S, 1), jnp.int32),
            jax.ShapeDtypeStruct((NBLK, 1), jnp.int32),
            jax.ShapeDtypeStruct((BN, E), jnp.float32),
        ],
        scratch_shapes=[pltpu.VMEM((S, E), jnp.float32)],
    )(idx_s, idx_t, cm_t)
    pos_rows = pos.reshape(NW, SPW)

    mesh = plsc.VectorSubcoreMesh(core_axis_name="c", subcore_axis_name="s",
                                  num_cores=NC, num_subcores=NS)

    @functools.partial(
        pl.kernel,
        out_type=jax.ShapeDtypeStruct((RMAX, T), jnp.float32),
        mesh=mesh,
        scratch_types=[
            pltpu.VMEM((SPW,), jnp.int32),
            pltpu.VMEM((SPW,), jnp.int32),
            pltpu.VMEM((SPW, T), jnp.float32),
            pltpu.SemaphoreType.DMA,
            pltpu.SemaphoreType.DMA,
        ],
    )
    def dispatch(x_hbm, tok_hbm, pos_hbm, xs_hbm, tok_v, pos_v, xbuf, s1, s2):
        wid = lax.axis_index("s") * NC + lax.axis_index("c")
        pltpu.sync_copy(tok_hbm.at[wid], tok_v)
        pltpu.sync_copy(pos_hbm.at[wid], pos_v)
        pltpu.async_copy(x_hbm.at[tok_v], xbuf, s1).wait()
        pltpu.async_copy(xbuf, xs_hbm.at[pos_v], s2).wait()

    xs = dispatch(x_flat, tok_of_slot, pos_rows)

    W1b, W2b = pl.pallas_call(
        _wcast_body,
        grid=(E,),
        in_specs=[
            pl.BlockSpec((1, T, H), lambda e: (e, 0, 0)),
            pl.BlockSpec((1, H, T), lambda e: (e, 0, 0)),
        ],
        out_specs=[
            pl.BlockSpec((1, T, H), lambda e: (e, 0, 0)),
            pl.BlockSpec((1, H, T), lambda e: (e, 0, 0)),
        ],
        out_shape=[
            jax.ShapeDtypeStruct((E, T, H), jnp.bfloat16),
            jax.ShapeDtypeStruct((E, H, T), jnp.bfloat16),
        ],
    )(W1, W2)

    BPS = 2
    z_sorted = pl.pallas_call(
        functools.partial(_mlp_body, BPS=BPS),
        grid_spec=pltpu.PrefetchScalarGridSpec(
            num_scalar_prefetch=1,
            grid=(NBLK // BPS,),
            in_specs=[
                pl.BlockSpec((BPS * _MB, T), lambda b, blk: (b, 0)),
                pl.BlockSpec((E, T, H), lambda b, blk: (0, 0, 0)),
                pl.BlockSpec((E, 1, H), lambda b, blk: (0, 0, 0)),
                pl.BlockSpec((E, H, T), lambda b, blk: (0, 0, 0)),
                pl.BlockSpec((E, 1, T), lambda b, blk: (0, 0, 0)),
            ],
            out_specs=pl.BlockSpec((BPS * _MB, T), lambda b, blk: (b, 0)),
        ),
        out_shape=jax.ShapeDtypeStruct((RMAX, T), jnp.float32),
        compiler_params=pltpu.CompilerParams(
            dimension_semantics=("arbitrary",),
        ),
    )(blk.reshape(NBLK), xs,
      W1b, b1.reshape(E, 1, H),
      W2b, b2.reshape(E, 1, T))

    TPW = BN // NW
    posK = pos.reshape(BN, K)
    pos_k = [posK[:, k].reshape(NW, TPW) for k in range(K)]

    @functools.partial(
        pl.kernel,
        out_type=[jax.ShapeDtypeStruct((BN, T), jnp.float32) for _ in range(K)],
        mesh=mesh,
        scratch_types=(
            [pltpu.VMEM((TPW,), jnp.int32) for _ in range(K)]
            + [pltpu.VMEM((TPW, T), jnp.float32) for _ in range(K)]
            + [pltpu.SemaphoreType.DMA]
        ),
    )
    def unsort(z_hbm, *refs):
        pos_hbms = refs[:K]
        z_outs = refs[K:2 * K]
        pos_vs = refs[2 * K:3 * K]
        zbufs = refs[3 * K:4 * K]
        s1 = refs[4 * K]
        wid = lax.axis_index("s") * NC + lax.axis_index("c")
        for k in range(K):
            pltpu.sync_copy(pos_hbms[k].at[wid], pos_vs[k])
            pltpu.async_copy(z_hbm.at[pos_vs[k]], zbufs[k], s1).wait()
            pltpu.sync_copy(zbufs[k], z_outs[k].at[pl.ds(wid * TPW, TPW)])

    z_k = unsort(z_sorted, *pos_k)

    TBc = 512
    out = pl.pallas_call(
        functools.partial(_combine_body, T=T),
        grid=(BN // TBc,),
        in_specs=(
            [
                pl.BlockSpec((TBc, T), lambda i: (i, 0)),
                pl.BlockSpec((TBc, K), lambda i: (i, 0)),
                pl.BlockSpec((TBc, K), lambda i: (i, 0)),
            ]
            + [pl.BlockSpec((TBc, T), lambda i: (i, 0)) for _ in range(K)]
            + [pl.BlockSpec(memory_space=pltpu.SMEM)]
        ),
        out_specs=pl.BlockSpec((TBc, T), lambda i: (i, 0)),
        out_shape=jax.ShapeDtypeStruct((BN, T), jnp.float32),
    )(x_flat, w_t, cm_t, *z_k, scale)

    return out.reshape(B, N, T), util.reshape(B, N, E)


# speedup vs baseline: 1.1522x; 1.0042x over previous
import functools

import jax
import jax.numpy as jnp
from jax import lax
from jax.experimental import pallas as pl
from jax.experimental.pallas import tpu as pltpu
from jax.experimental.pallas import tpu_sc as plsc

_MB = 256
_CH = 512


def _meta_body(idx_s_ref, idx_t_ref, cm_t_ref, pos_ref, blk_ref, util_ref,
               rank_ref, *, S, E, NBLK):
    eids = lax.broadcasted_iota(jnp.int32, (S, E), 1)
    O = (idx_s_ref[...] == eids).astype(jnp.float32)

    ri = lax.broadcasted_iota(jnp.int32, (_CH, _CH), 0)
    ci = lax.broadcasted_iota(jnp.int32, (_CH, _CH), 1)
    LT = (ri > ci).astype(jnp.float32)
    carry = jnp.zeros((1, E), jnp.float32)
    for i in range(S // _CH):
        Oc = O[i * _CH:(i + 1) * _CH]
        rank_ref[i * _CH:(i + 1) * _CH, :] = (
            jnp.dot(LT, Oc, preferred_element_type=jnp.float32) + carry)
        carry = carry + jnp.sum(Oc, axis=0, keepdims=True)
    counts = carry
    padded = jnp.floor((counts + (_MB - 1)) * (1.0 / _MB)) * _MB

    eri = lax.broadcasted_iota(jnp.int32, (E, E), 0)
    eci = lax.broadcasted_iota(jnp.int32, (E, E), 1)
    UT = (eri < eci).astype(jnp.float32)
    bstart = jnp.dot(padded, UT, preferred_element_type=jnp.float32)

    posf = jnp.sum((rank_ref[...] + bstart) * O, axis=1, keepdims=True)
    pos_ref[...] = posf.astype(jnp.int32)

    bpos = lax.broadcasted_iota(jnp.int32, (NBLK, 1), 0).astype(jnp.float32) * _MB
    cnt = jnp.sum((bstart <= bpos).astype(jnp.float32), axis=1, keepdims=True)
    total = jnp.sum(padded)
    blk_ref[...] = jnp.where(bpos < total, cnt - 1.0, -1.0).astype(jnp.int32)

    eids_t = lax.broadcasted_iota(jnp.int32, (idx_t_ref.shape[0], E), 1)
    acc = jnp.zeros((idx_t_ref.shape[0], E), jnp.float32)
    for k in range(idx_t_ref.shape[1]):
        hit = (idx_t_ref[:, k:k + 1] == eids_t) & (cm_t_ref[:, k:k + 1] > 0.0)
        acc = jnp.maximum(acc, hit.astype(jnp.float32))
    util_ref[...] = acc


def _wcast_body(W1_ref, W2_ref, W1o_ref, W2o_ref):
    W1o_ref[...] = W1_ref[...].astype(jnp.bfloat16)
    W2o_ref[...] = W2_ref[...].astype(jnp.bfloat16)


def _mlp_body(blk_ref, xs_ref, W1_ref, b1_ref, W2_ref, b2_ref, z_ref, *, BPS):
    b = pl.program_id(0)
    for half in range(BPS):
        blk_id = blk_ref[b * BPS + half]
        lo = half * _MB

        @pl.when(blk_id >= 0)
        def _(blk_id=blk_id, lo=lo):
            e = jnp.maximum(blk_id, 0)
            xb = xs_ref[lo:lo + _MB, :].astype(jnp.bfloat16)
            h = jnp.maximum(
                jnp.dot(xb, W1_ref[e], preferred_element_type=jnp.float32)
                + b1_ref[e], 0.0)
            z_ref[lo:lo + _MB, :] = jnp.dot(
                h.astype(jnp.bfloat16), W2_ref[e],
                preferred_element_type=jnp.float32) + b2_ref[e]

        @pl.when(blk_id < 0)
        def _(lo=lo):
            z_ref[lo:lo + _MB, :] = jnp.zeros((_MB, z_ref.shape[1]), z_ref.dtype)


def _combine_body(x_ref, w_ref, cm_ref, *rest, T):
    K = w_ref.shape[1]
    z_refs = rest[:K]
    scale_ref = rest[K]
    out_ref = rest[K + 1]
    wb = w_ref[...]
    cmb = cm_ref[...]
    scale = scale_ref[0, 0]
    wsum = jnp.clip(jnp.sum(wb, axis=1, keepdims=True), 1e-12, None)
    nw = wb / wsum
    acc = jnp.sum(nw, axis=1, keepdims=True) * x_ref[...]
    for k in range(K):
        acc = acc + (nw[:, k:k + 1] * cmb[:, k:k + 1]) * z_refs[k][...]
    out_ref[...] = scale * acc


def kernel(x, chosen_idx, chosen_w, compute_mask, route_scale, W1, b1, W2, b2):
    B, N, T = x.shape
    K = chosen_idx.shape[-1]
    E, _, H = W1.shape
    BN = B * N
    S = BN * K
    NBLK = S // _MB + E
    RMAX = NBLK * _MB
    NC, NS = 2, 16
    NW = NC * NS
    SPW = S // NW

    x_flat = x.reshape(BN, T)
    idx_s = chosen_idx.reshape(S, 1).astype(jnp.int32)
    idx_t = chosen_idx.reshape(BN, K).astype(jnp.int32)
    w_t = chosen_w.reshape(BN, K)
    cm_t = compute_mask.reshape(BN, K).astype(jnp.float32)
    scale = route_scale.reshape(1, 1).astype(jnp.float32)
    tok_of_slot = (jnp.arange(S, dtype=jnp.int32) // K).reshape(NW, SPW)

    pos, blk, util = pl.pallas_call(
        functools.partial(_meta_body, S=S, E=E, NBLK=NBLK),
        out_shape=[
            jax.ShapeDtypeStruct((S, 1), jnp.int32),
            jax.ShapeDtypeStruct((NBLK, 1), jnp.int32),
            jax.ShapeDtypeStruct((BN, E), jnp.float32),
        ],
        scratch_shapes=[pltpu.VMEM((S, E), jnp.float32)],
    )(idx_s, idx_t, cm_t)
    pos_rows = pos.reshape(NW, SPW)

    mesh = plsc.VectorSubcoreMesh(core_axis_name="c", subcore_axis_name="s",
                                  num_cores=NC, num_subcores=NS)

    @functools.partial(
        pl.kernel,
        out_type=jax.ShapeDtypeStruct((RMAX, T), jnp.float32),
        mesh=mesh,
        scratch_types=[
            pltpu.VMEM((SPW,), jnp.int32),
            pltpu.VMEM((SPW,), jnp.int32),
            pltpu.VMEM((SPW, T), jnp.float32),
            pltpu.SemaphoreType.DMA,
            pltpu.SemaphoreType.DMA,
        ],
    )
    def dispatch(x_hbm, tok_hbm, pos_hbm, xs_hbm, tok_v, pos_v, xbuf, s1, s2):
        wid = lax.axis_index("s") * NC + lax.axis_index("c")
        pltpu.sync_copy(tok_hbm.at[wid], tok_v)
        pltpu.sync_copy(pos_hbm.at[wid], pos_v)
        pltpu.async_copy(x_hbm.at[tok_v], xbuf, s1).wait()
        pltpu.async_copy(xbuf, xs_hbm.at[pos_v], s2).wait()

    xs = dispatch(x_flat, tok_of_slot, pos_rows)

    W1b, W2b = pl.pallas_call(
        _wcast_body,
        grid=(E,),
        in_specs=[
            pl.BlockSpec((1, T, H), lambda e: (e, 0, 0)),
            pl.BlockSpec((1, H, T), lambda e: (e, 0, 0)),
        ],
        out_specs=[
            pl.BlockSpec((1, T, H), lambda e: (e, 0, 0)),
            pl.BlockSpec((1, H, T), lambda e: (e, 0, 0)),
        ],
        out_shape=[
            jax.ShapeDtypeStruct((E, T, H), jnp.bfloat16),
            jax.ShapeDtypeStruct((E, H, T), jnp.bfloat16),
        ],
    )(W1, W2)

    BPS = 4
    z_sorted = pl.pallas_call(
        functools.partial(_mlp_body, BPS=BPS),
        grid_spec=pltpu.PrefetchScalarGridSpec(
            num_scalar_prefetch=1,
            grid=(NBLK // BPS,),
            in_specs=[
                pl.BlockSpec((BPS * _MB, T), lambda b, blk: (b, 0)),
                pl.BlockSpec((E, T, H), lambda b, blk: (0, 0, 0)),
                pl.BlockSpec((E, 1, H), lambda b, blk: (0, 0, 0)),
                pl.BlockSpec((E, H, T), lambda b, blk: (0, 0, 0)),
                pl.BlockSpec((E, 1, T), lambda b, blk: (0, 0, 0)),
            ],
            out_specs=pl.BlockSpec((BPS * _MB, T), lambda b, blk: (b, 0)),
        ),
        out_shape=jax.ShapeDtypeStruct((RMAX, T), jnp.float32),
        compiler_params=pltpu.CompilerParams(
            dimension_semantics=("arbitrary",),
        ),
    )(blk.reshape(NBLK), xs,
      W1b, b1.reshape(E, 1, H),
      W2b, b2.reshape(E, 1, T))

    TPW = BN // NW
    posK = pos.reshape(BN, K)
    pos_k = [posK[:, k].reshape(NW, TPW) for k in range(K)]

    @functools.partial(
        pl.kernel,
        out_type=[jax.ShapeDtypeStruct((BN, T), jnp.float32) for _ in range(K)],
        mesh=mesh,
        scratch_types=(
            [pltpu.VMEM((TPW,), jnp.int32) for _ in range(K)]
            + [pltpu.VMEM((TPW, T), jnp.float32) for _ in range(K)]
            + [pltpu.SemaphoreType.DMA]
        ),
    )
    def unsort(z_hbm, *refs):
        pos_hbms = refs[:K]
        z_outs = refs[K:2 * K]
        pos_vs = refs[2 * K:3 * K]
        zbufs = refs[3 * K:4 * K]
        s1 = refs[4 * K]
        wid = lax.axis_index("s") * NC + lax.axis_index("c")
        for k in range(K):
            pltpu.sync_copy(pos_hbms[k].at[wid], pos_vs[k])
            pltpu.async_copy(z_hbm.at[pos_vs[k]], zbufs[k], s1).wait()
            pltpu.sync_copy(zbufs[k], z_outs[k].at[pl.ds(wid * TPW, TPW)])

    z_k = unsort(z_sorted, *pos_k)

    TBc = 512
    out = pl.pallas_call(
        functools.partial(_combine_body, T=T),
        grid=(BN // TBc,),
        in_specs=(
            [
                pl.BlockSpec((TBc, T), lambda i: (i, 0)),
                pl.BlockSpec((TBc, K), lambda i: (i, 0)),
                pl.BlockSpec((TBc, K), lambda i: (i, 0)),
            ]
            + [pl.BlockSpec((TBc, T), lambda i: (i, 0)) for _ in range(K)]
            + [pl.BlockSpec(memory_space=pltpu.SMEM)]
        ),
        out_specs=pl.BlockSpec((TBc, T), lambda i: (i, 0)),
        out_shape=jax.ShapeDtypeStruct((BN, T), jnp.float32),
    )(x_flat, w_t, cm_t, *z_k, scale)

    return out.reshape(B, N, T), util.reshape(B, N, E)
